# baseline (speedup 1.0000x reference)
import jax
import jax.numpy as jnp
from jax import lax
from jax.experimental import pallas as pl
from jax.experimental.pallas import tpu as pltpu
from jax.experimental.pallas import tpu_sc as plsc

N_CLS = 1000
N_CTX = 16
D = 768
CTX_LEN = 77
SUF = CTX_LEN - 1 - N_CTX
NW = 32
PER_W = 32


def _body(ctx_hbm, prefix_hbm, suffix_hbm, out_hbm, ctx_v):
    wid = lax.axis_index("s") * 2 + lax.axis_index("c")
    pltpu.sync_copy(ctx_hbm, ctx_v)

    def cls_body(i, carry):
        c = wid * PER_W + i

        @pl.when(c < N_CLS)
        def _do():
            pltpu.sync_copy(prefix_hbm.at[c], out_hbm.at[c, pl.ds(0, 1)])
            pltpu.sync_copy(ctx_v, out_hbm.at[c, pl.ds(1, N_CTX)])
            pltpu.sync_copy(suffix_hbm.at[c], out_hbm.at[c, pl.ds(1 + N_CTX, SUF)])

        return carry

    lax.fori_loop(0, PER_W, cls_body, 0)


def kernel(ctx, prefix_embedding, suffix_embedding):
    mesh = plsc.VectorSubcoreMesh(core_axis_name="c", subcore_axis_name="s")
    k = pl.kernel(
        _body,
        out_type=jax.ShapeDtypeStruct((N_CLS, CTX_LEN, D), jnp.float32),
        mesh=mesh,
        scratch_types=[pltpu.VMEM((N_CTX, D), jnp.float32)],
        compiler_params=pltpu.CompilerParams(use_tc_tiling_on_sc=False),
    )
    return k(ctx, prefix_embedding, suffix_embedding)


# speedup vs baseline: 10.9934x; 10.9934x over previous
import jax
import jax.numpy as jnp
from jax import lax
from jax.experimental import pallas as pl
from jax.experimental.pallas import tpu as pltpu
from jax.experimental.pallas import tpu_sc as plsc

N_CLS = 1000
N_CTX = 16
D = 768
CTX_LEN = 77
SUF = CTX_LEN - 1 - N_CTX
SUF_MAIN = 56
TAIL = 8
NW = 32
PER_W = 32


def _body(
    ctx_hbm,
    prefix_hbm,
    suffix_hbm,
    suffix_flat_hbm,
    idx_ctx_hbm,
    idx_suf_hbm,
    idx_tail_even_hbm,
    idx_tail_odd_hbm,
    out_hbm,
    ctx_v,
    suf_v,
    tail_v,
    idx_ctx_v,
    idx_suf_v,
    idx_tail_even_v,
    idx_tail_odd_v,
):
    wid = lax.axis_index("s") * 2 + lax.axis_index("c")
    pltpu.sync_copy(ctx_hbm, ctx_v)
    pltpu.sync_copy(idx_ctx_hbm, idx_ctx_v)
    pltpu.sync_copy(idx_suf_hbm, idx_suf_v)
    pltpu.sync_copy(idx_tail_even_hbm, idx_tail_even_v)
    pltpu.sync_copy(idx_tail_odd_hbm, idx_tail_odd_v)

    def cls_body(i, carry):
        c = wid * PER_W + i

        @pl.when(c < N_CLS)
        def _do():
            out_c = out_hbm.at[c]
            pltpu.sync_copy(suffix_hbm.at[c].at[pl.ds(0, SUF_MAIN)], suf_v)
            pltpu.sync_copy(suf_v, out_c.at[idx_suf_v])

            is_odd = lax.rem(c, 2)
            off = pl.multiple_of(c * SUF + SUF_MAIN - 4 * is_odd, 8)
            pltpu.sync_copy(suffix_flat_hbm.at[pl.ds(off, TAIL)], tail_v)

            @pl.when(is_odd == 0)
            def _even():
                pltpu.sync_copy(tail_v, out_c.at[idx_tail_even_v])

            @pl.when(is_odd == 1)
            def _odd():
                pltpu.sync_copy(tail_v, out_c.at[idx_tail_odd_v])

            pltpu.sync_copy(ctx_v, out_c.at[idx_ctx_v])
            pltpu.sync_copy(prefix_hbm.at[c], out_c.at[pl.ds(0, 1)])

        return carry

    lax.fori_loop(0, PER_W, cls_body, 0)


def kernel(ctx, prefix_embedding, suffix_embedding):
    suffix_flat = suffix_embedding.reshape(N_CLS * SUF, D)
    idx_ctx = jnp.arange(1, 1 + N_CTX, dtype=jnp.int32)
    idx_suf = jnp.arange(1 + N_CTX, 1 + N_CTX + SUF_MAIN, dtype=jnp.int32)
    idx_tail_even = jnp.array([73, 74, 75, 76, 1, 2, 3, 4], dtype=jnp.int32)
    idx_tail_odd = jnp.arange(CTX_LEN - TAIL, CTX_LEN, dtype=jnp.int32)
    mesh = plsc.VectorSubcoreMesh(core_axis_name="c", subcore_axis_name="s")
    k = pl.kernel(
        _body,
        out_type=jax.ShapeDtypeStruct((N_CLS, CTX_LEN, D), jnp.float32),
        mesh=mesh,
        scratch_types=[
            pltpu.VMEM((N_CTX, D), jnp.float32),
            pltpu.VMEM((SUF_MAIN, D), jnp.float32),
            pltpu.VMEM((TAIL, D), jnp.float32),
            pltpu.VMEM((N_CTX,), jnp.int32),
            pltpu.VMEM((SUF_MAIN,), jnp.int32),
            pltpu.VMEM((TAIL,), jnp.int32),
            pltpu.VMEM((TAIL,), jnp.int32),
        ],
    )
    return k(
        ctx,
        prefix_embedding,
        suffix_embedding,
        suffix_flat,
        idx_ctx,
        idx_suf,
        idx_tail_even,
        idx_tail_odd,
    )
